# Initial kernel scaffold; baseline (speedup 1.0000x reference)
import jax
import jax.numpy as jnp
from jax.experimental import pallas as pl


def kernel(x, edge_index, W1, b1, W2, b2, W3, b3):
    raise NotImplementedError("write your pallas kernel here")


# speedup vs baseline: 5.3106x; 5.3106x over previous
import functools

import jax
import jax.numpy as jnp
from jax import lax
from jax.experimental import pallas as pl
from jax.experimental.pallas import tpu as pltpu
from jax.experimental.pallas import tpu_sc as plsc

NC = 2
NS = 16
F = 128
CH = 128


def _round_up(a, b):
    return (a + b - 1) // b * b


def _sc_mesh():
    return plsc.VectorSubcoreMesh(
        core_axis_name="c", subcore_axis_name="s", num_cores=NC, num_subcores=NS
    )


def _make_deg_kernel(n_pad, e_pad):
    rt = n_pad // NS
    cpw = e_pad // (NC * NS * CH)

    def body(col_hbm, ones_hbm, zeros_hbm, out_hbm, idx_v, ones_v, shared):
        c = lax.axis_index("c")
        t = lax.axis_index("s")
        w = c * NS + t
        pltpu.sync_copy(col_hbm.at[w], idx_v)
        pltpu.sync_copy(ones_hbm, ones_v)
        pltpu.sync_copy(zeros_hbm, shared.at[pl.ds(t * rt, rt)])
        plsc.subcore_barrier()

        def chunk(i, _):
            pltpu.sync_copy(ones_v, shared.at[idx_v.at[i]], add=True)
            return 0

        lax.fori_loop(0, cpw, chunk, 0)
        plsc.subcore_barrier()
        pltpu.sync_copy(
            shared.at[pl.ds(t * rt, rt)],
            out_hbm.at[pl.ds(c * n_pad + t * rt, rt)],
        )

    return pl.kernel(
        body,
        out_type=jax.ShapeDtypeStruct((NC * n_pad, F), jnp.float32),
        mesh=_sc_mesh(),
        scratch_types=[
            pltpu.VMEM((cpw, CH), jnp.int32),
            pltpu.VMEM((CH, F), jnp.float32),
            pltpu.VMEM_SHARED((n_pad, F), jnp.float32),
        ],
    )


def _make_agg_kernel(n_pad, e_pad, n_slices):
    rt = n_pad // NS
    cpt = e_pad // (NS * CH)
    spc = n_slices // NC

    def body(g_hbm, row_hbm, col_hbm, zeros_hbm, out_hbm, idx_r, idx_c, buf, shared):
        c = lax.axis_index("c")
        t = lax.axis_index("s")
        pltpu.sync_copy(col_hbm.at[t], idx_c)
        for j in range(spc):
            s = c * spc + j
            pltpu.sync_copy(row_hbm.at[s * NS + t], idx_r)
            pltpu.sync_copy(zeros_hbm, shared.at[pl.ds(t * rt, rt)])
            plsc.subcore_barrier()

            def chunk(i, _):
                pltpu.sync_copy(g_hbm.at[idx_r.at[i]], buf)
                pltpu.sync_copy(buf, shared.at[idx_c.at[i]], add=True)
                return 0

            lax.fori_loop(0, cpt, chunk, 0)
            plsc.subcore_barrier()
            pltpu.sync_copy(
                shared.at[pl.ds(t * rt, rt)],
                out_hbm.at[pl.ds(s * n_pad + t * rt, rt)],
            )

    return pl.kernel(
        body,
        out_type=jax.ShapeDtypeStruct((n_slices * n_pad, F), jnp.float32),
        mesh=_sc_mesh(),
        scratch_types=[
            pltpu.VMEM((cpt, CH), jnp.int32),
            pltpu.VMEM((cpt, CH), jnp.int32),
            pltpu.VMEM((CH, F), jnp.float32),
            pltpu.VMEM_SHARED((n_pad, F), jnp.float32),
        ],
    )


def _dis_from_deg(deg_ref):
    deg = deg_ref[0, :, 0:1] + deg_ref[1, :, 0:1] + 1.0
    return lax.rsqrt(deg)


def _mm1_body(x_ref, w_ref, b_ref, deg_ref, out_ref):
    dis = _dis_from_deg(deg_ref)
    x = x_ref[...]
    w = w_ref[...]
    b = b_ref[...]
    ns = out_ref.shape[0]
    for s in range(ns):
        h = lax.dot_general(
            x, w[s * F:(s + 1) * F, :], (((1,), (1,)), ((), ())),
            preferred_element_type=jnp.float32,
        )
        out_ref[s] = dis * (h + b[:, s * F:(s + 1) * F])


def _mm2_body(agg_ref, g_ref, w_ref, b_ref, deg_ref, out_ref):
    dis = _dis_from_deg(deg_ref)
    w = w_ref[...]
    b = b_ref[...]
    ns = out_ref.shape[0]
    hs = [jax.nn.relu(dis * (agg_ref[k] + g_ref[k])) for k in range(ns)]
    for s in range(ns):
        acc = None
        for k in range(ns):
            p = lax.dot_general(
                hs[k], w[s * F:(s + 1) * F, k * F:(k + 1) * F],
                (((1,), (1,)), ((), ())), preferred_element_type=jnp.float32,
            )
            acc = p if acc is None else acc + p
        out_ref[s] = dis * (acc + b[:, s * F:(s + 1) * F])


def _mm3_body(agg_ref, g_ref, w_ref, b_ref, deg_ref, out_ref):
    dis = _dis_from_deg(deg_ref)
    w = w_ref[...]
    ns = agg_ref.shape[0]
    acc = None
    for k in range(ns):
        h = jax.nn.relu(dis * (agg_ref[k] + g_ref[k]))
        p = lax.dot_general(
            h, w[:, k * F:(k + 1) * F], (((1,), (1,)), ((), ())),
            preferred_element_type=jnp.float32,
        )
        acc = p if acc is None else acc + p
    out_ref[...] = acc + b_ref[...]


def kernel(x, edge_index, W1, b1, W2, b2, W3, b3):
    n, d_in = x.shape
    e = edge_index.shape[1]
    d_h = W1.shape[0]
    d_out = W3.shape[0]
    ns = d_h // F

    n_pad = _round_up(n, NS * CH)
    e_pad = _round_up(e, NC * NS * CH)
    rt = n_pad // NS
    br = 1024
    nr = n_pad // br

    f32 = jnp.float32
    row = edge_index[0]
    col = edge_index[1]
    row_pad = jnp.concatenate([row, jnp.zeros((e_pad - e,), jnp.int32)])
    col_pad = jnp.concatenate(
        [col, jnp.full((e_pad - e,), n_pad - 1, jnp.int32)]
    )
    row_r = row_pad.reshape(1, NS, e_pad // (NS * CH), CH)
    offs = (jnp.arange(ns, dtype=jnp.int32) * n_pad).reshape(ns, 1, 1, 1)
    row_all = (row_r + offs).reshape(ns * NS, e_pad // (NS * CH), CH)
    col_agg = col_pad.reshape(NS, e_pad // (NS * CH), CH)
    col_deg = col_pad.reshape(NC * NS, e_pad // (NC * NS * CH), CH)

    x_pad = jnp.pad(x, ((0, n_pad - n), (0, 0)))
    zeros_f = jnp.zeros((rt, F), f32)
    ones_f = jnp.ones((CH, F), f32)
    b1r = b1.reshape(1, d_h)
    b2r = b2.reshape(1, d_h)
    b3r = b3.reshape(1, d_out)

    deg2 = _make_deg_kernel(n_pad, e_pad)(col_deg, ones_f, zeros_f)
    deg2 = deg2.reshape(NC, n_pad, F)

    agg_fn = _make_agg_kernel(n_pad, e_pad, ns)

    deg_spec = pl.BlockSpec((NC, br, F), lambda i: (0, i, 0))
    sl_spec = pl.BlockSpec((ns, br, F), lambda i: (0, i, 0))

    g1 = pl.pallas_call(
        _mm1_body,
        grid=(nr,),
        in_specs=[
            pl.BlockSpec((br, d_in), lambda i: (i, 0)),
            pl.BlockSpec((d_h, d_in), lambda i: (0, 0)),
            pl.BlockSpec((1, d_h), lambda i: (0, 0)),
            deg_spec,
        ],
        out_specs=sl_spec,
        out_shape=jax.ShapeDtypeStruct((ns, n_pad, F), f32),
    )(x_pad, W1, b1r, deg2)

    agg1 = agg_fn(
        g1.reshape(ns * n_pad, F), row_all, col_agg, zeros_f
    ).reshape(ns, n_pad, F)

    g2 = pl.pallas_call(
        _mm2_body,
        grid=(nr,),
        in_specs=[
            sl_spec,
            sl_spec,
            pl.BlockSpec((d_h, d_h), lambda i: (0, 0)),
            pl.BlockSpec((1, d_h), lambda i: (0, 0)),
            deg_spec,
        ],
        out_specs=sl_spec,
        out_shape=jax.ShapeDtypeStruct((ns, n_pad, F), f32),
    )(agg1, g1, W2, b2r, deg2)

    agg2 = agg_fn(
        g2.reshape(ns * n_pad, F), row_all, col_agg, zeros_f
    ).reshape(ns, n_pad, F)

    out = pl.pallas_call(
        _mm3_body,
        grid=(nr,),
        in_specs=[
            sl_spec,
            sl_spec,
            pl.BlockSpec((d_out, d_h), lambda i: (0, 0)),
            pl.BlockSpec((1, d_out), lambda i: (0, 0)),
            deg_spec,
        ],
        out_specs=pl.BlockSpec((br, d_out), lambda i: (i, 0)),
        out_shape=jax.ShapeDtypeStruct((n_pad, d_out), f32),
    )(agg2, g2, W3, b3r, deg2)

    return out[:n]
